# initial kernel scaffold (unmeasured)
import jax
import jax.numpy as jnp
from jax import lax
from jax.experimental import pallas as pl
from jax.experimental.pallas import tpu as pltpu

N_DEV = 4
B, SQ, H, D = 4, 32, 8, 128
KV_CHUNK = 512
SCALE = D ** -0.5
NEG_INF = -1e30


def _flash_partial_body(q_ref, k_ref, v_ref, o_ref, st_ref, m_s, l_s, acc):
    c = pl.program_id(1)
    nc = pl.num_programs(1)

    @pl.when(c == 0)
    def _():
        m_s[...] = jnp.full((B, SQ, 1), NEG_INF, jnp.float32)
        l_s[...] = jnp.zeros((B, SQ, 1), jnp.float32)
        acc[...] = jnp.zeros((B, SQ, D), jnp.float32)

    for b in range(B):
        q = q_ref[b] * SCALE
        k = k_ref[b]
        v = v_ref[b]
        s = lax.dot_general(
            q, k, (((1,), (1,)), ((), ())),
            preferred_element_type=jnp.float32,
        )
        m_old = m_s[b]
        m_new = jnp.maximum(m_old, jnp.max(s, axis=1, keepdims=True))
        p = jnp.exp(s - m_new)
        alpha = jnp.exp(m_old - m_new)
        l_s[b] = alpha * l_s[b] + jnp.sum(p, axis=1, keepdims=True)
        pv = lax.dot_general(
            p, v, (((1,), (0,)), ((), ())),
            preferred_element_type=jnp.float32,
        )
        acc[b] = alpha * acc[b] + pv
        m_s[b] = m_new

    @pl.when(c == nc - 1)
    def _():
        for b in range(B):
            o_ref[b] = acc[b]
            st_ref[b] = jnp.concatenate([m_s[b], l_s[b]], axis=1)


def _flash_partial(Q, K, V):
    skv = K.shape[1]
    nc = skv // KV_CHUNK
    return pl.pallas_call(
        _flash_partial_body,
        grid=(H, nc),
        in_specs=[
            pl.BlockSpec((B, SQ, None, D), lambda h, c: (0, 0, h, 0)),
            pl.BlockSpec((B, KV_CHUNK, None, D), lambda h, c: (0, c, h, 0)),
            pl.BlockSpec((B, KV_CHUNK, None, D), lambda h, c: (0, c, h, 0)),
        ],
        out_specs=[
            pl.BlockSpec((B, None, SQ, D), lambda h, c: (0, h, 0, 0)),
            pl.BlockSpec((B, None, SQ, 2), lambda h, c: (0, h, 0, 0)),
        ],
        out_shape=[
            jax.ShapeDtypeStruct((B, H, SQ, D), jnp.float32),
            jax.ShapeDtypeStruct((B, H, SQ, 2), jnp.float32),
        ],
        scratch_shapes=[
            pltpu.VMEM((B, SQ, 1), jnp.float32),
            pltpu.VMEM((B, SQ, 1), jnp.float32),
            pltpu.VMEM((B, SQ, D), jnp.float32),
        ],
    )(Q, K, V)


def _allreduce_body(o_ref, st_ref, out_ref, o_comm, s_comm,
                    so_sems, ro_sems, ss_sems, rs_sems):
    my = lax.axis_index("i")
    left = lax.rem(my + N_DEV - 1, N_DEV)
    right = lax.rem(my + 1, N_DEV)

    o_comm[0] = o_ref[...]
    s_comm[0] = st_ref[...]

    barrier_sem = pltpu.get_barrier_semaphore()
    for nbr in (left, right):
        pl.semaphore_signal(
            barrier_sem, inc=1,
            device_id=(nbr,), device_id_type=pl.DeviceIdType.MESH,
        )
    pl.semaphore_wait(barrier_sem, 2)

    for h in range(N_DEV - 1):
        rdma_o = pltpu.make_async_remote_copy(
            src_ref=o_comm.at[h],
            dst_ref=o_comm.at[h + 1],
            send_sem=so_sems.at[h],
            recv_sem=ro_sems.at[h],
            device_id=(right,),
            device_id_type=pl.DeviceIdType.MESH,
        )
        rdma_s = pltpu.make_async_remote_copy(
            src_ref=s_comm.at[h],
            dst_ref=s_comm.at[h + 1],
            send_sem=ss_sems.at[h],
            recv_sem=rs_sems.at[h],
            device_id=(right,),
            device_id_type=pl.DeviceIdType.MESH,
        )
        rdma_o.start()
        rdma_s.start()
        rdma_o.wait()
        rdma_s.wait()

    for b in range(B):
        for hh in range(H):
            ms = [s_comm[j, b, hh, :, 0:1] for j in range(N_DEV)]
            m_tot = ms[0]
            for j in range(1, N_DEV):
                m_tot = jnp.maximum(m_tot, ms[j])
            l_tot = jnp.zeros((SQ, 1), jnp.float32)
            o_tot = jnp.zeros((SQ, D), jnp.float32)
            for j in range(N_DEV):
                w = jnp.exp(ms[j] - m_tot)
                l_tot = l_tot + w * s_comm[j, b, hh, :, 1:2]
                o_tot = o_tot + w * o_comm[j, b, hh]
            out_ref[b, :, hh, :] = o_tot / l_tot


def _allreduce_combine(o_part, stats):
    return pl.pallas_call(
        _allreduce_body,
        in_specs=[
            pl.BlockSpec(memory_space=pltpu.VMEM),
            pl.BlockSpec(memory_space=pltpu.VMEM),
        ],
        out_specs=pl.BlockSpec(memory_space=pltpu.VMEM),
        out_shape=jax.ShapeDtypeStruct((B, SQ, H, D), jnp.float32),
        scratch_shapes=[
            pltpu.VMEM((N_DEV, B, H, SQ, D), jnp.float32),
            pltpu.VMEM((N_DEV, B, H, SQ, 2), jnp.float32),
            pltpu.SemaphoreType.DMA((N_DEV - 1,)),
            pltpu.SemaphoreType.DMA((N_DEV - 1,)),
            pltpu.SemaphoreType.DMA((N_DEV - 1,)),
            pltpu.SemaphoreType.DMA((N_DEV - 1,)),
        ],
        compiler_params=pltpu.CompilerParams(collective_id=0),
    )(o_part, stats)


def kernel(Q, K, V):
    o_part, stats = _flash_partial(Q, K, V)
    return _allreduce_combine(o_part, stats)


# baseline (device time: 210067 ns/iter reference)
import jax
import jax.numpy as jnp
from jax import lax
from jax.experimental import pallas as pl
from jax.experimental.pallas import tpu as pltpu

N_DEV = 4
B, SQ, H, D = 4, 32, 8, 128
KV_CHUNK = 256
SCALE = D ** -0.5
NEG_INF = -1e30


def _flash_partial_body(q_ref, k_ref, v_ref, o_ref, st_ref, m_s, l_s, acc):
    c = pl.program_id(0)
    nc = pl.num_programs(0)

    @pl.when(c == 0)
    def _():
        m_s[...] = jnp.full((B, H, SQ, 1), NEG_INF, jnp.float32)
        l_s[...] = jnp.zeros((B, H, SQ, 1), jnp.float32)
        acc[...] = jnp.zeros((B, H, SQ, D), jnp.float32)

    for b in range(B):
        for h in range(H):
            q = q_ref[b, :, h, :] * SCALE
            k = k_ref[b, :, h, :]
            v = v_ref[b, :, h, :]
            s = lax.dot_general(
                q, k, (((1,), (1,)), ((), ())),
                preferred_element_type=jnp.float32,
            )
            m_old = m_s[b, h]
            m_new = jnp.maximum(m_old, jnp.max(s, axis=1, keepdims=True))
            p = jnp.exp(s - m_new)
            alpha = jnp.exp(m_old - m_new)
            l_s[b, h] = alpha * l_s[b, h] + jnp.sum(p, axis=1, keepdims=True)
            pv = lax.dot_general(
                p, v, (((1,), (0,)), ((), ())),
                preferred_element_type=jnp.float32,
            )
            acc[b, h] = alpha * acc[b, h] + pv
            m_s[b, h] = m_new

    @pl.when(c == nc - 1)
    def _():
        for b in range(B):
            for h in range(H):
                o_ref[b, h] = acc[b, h]
                st_ref[b, h] = jnp.concatenate(
                    [m_s[b, h], l_s[b, h]], axis=1
                )


def _flash_partial(Q, K, V):
    skv = K.shape[1]
    nc = skv // KV_CHUNK
    return pl.pallas_call(
        _flash_partial_body,
        grid=(nc,),
        in_specs=[
            pl.BlockSpec((B, SQ, H, D), lambda c: (0, 0, 0, 0)),
            pl.BlockSpec((B, KV_CHUNK, H, D), lambda c: (0, c, 0, 0)),
            pl.BlockSpec((B, KV_CHUNK, H, D), lambda c: (0, c, 0, 0)),
        ],
        out_specs=[
            pl.BlockSpec((B, H, SQ, D), lambda c: (0, 0, 0, 0)),
            pl.BlockSpec((B, H, SQ, 2), lambda c: (0, 0, 0, 0)),
        ],
        out_shape=[
            jax.ShapeDtypeStruct((B, H, SQ, D), jnp.float32),
            jax.ShapeDtypeStruct((B, H, SQ, 2), jnp.float32),
        ],
        scratch_shapes=[
            pltpu.VMEM((B, H, SQ, 1), jnp.float32),
            pltpu.VMEM((B, H, SQ, 1), jnp.float32),
            pltpu.VMEM((B, H, SQ, D), jnp.float32),
        ],
    )(Q, K, V)


def _allreduce_body(o_ref, st_ref, out_ref, o_comm, s_comm,
                    so_sems, ro_sems, ss_sems, rs_sems):
    my = lax.axis_index("i")
    left = lax.rem(my + N_DEV - 1, N_DEV)
    right = lax.rem(my + 1, N_DEV)

    o_comm[0] = o_ref[...]
    s_comm[0] = st_ref[...]

    barrier_sem = pltpu.get_barrier_semaphore()
    for nbr in (left, right):
        pl.semaphore_signal(
            barrier_sem, inc=1,
            device_id=(nbr,), device_id_type=pl.DeviceIdType.MESH,
        )
    pl.semaphore_wait(barrier_sem, 2)

    for h in range(N_DEV - 1):
        rdma_o = pltpu.make_async_remote_copy(
            src_ref=o_comm.at[h],
            dst_ref=o_comm.at[h + 1],
            send_sem=so_sems.at[h],
            recv_sem=ro_sems.at[h],
            device_id=(right,),
            device_id_type=pl.DeviceIdType.MESH,
        )
        rdma_s = pltpu.make_async_remote_copy(
            src_ref=s_comm.at[h],
            dst_ref=s_comm.at[h + 1],
            send_sem=ss_sems.at[h],
            recv_sem=rs_sems.at[h],
            device_id=(right,),
            device_id_type=pl.DeviceIdType.MESH,
        )
        rdma_o.start()
        rdma_s.start()
        rdma_o.wait()
        rdma_s.wait()

    for b in range(B):
        for hh in range(H):
            ms = [s_comm[j, b, hh, :, 0:1] for j in range(N_DEV)]
            m_tot = ms[0]
            for j in range(1, N_DEV):
                m_tot = jnp.maximum(m_tot, ms[j])
            l_tot = jnp.zeros((SQ, 1), jnp.float32)
            o_tot = jnp.zeros((SQ, D), jnp.float32)
            for j in range(N_DEV):
                w = jnp.exp(ms[j] - m_tot)
                l_tot = l_tot + w * s_comm[j, b, hh, :, 1:2]
                o_tot = o_tot + w * o_comm[j, b, hh]
            out_ref[b, :, hh, :] = o_tot / l_tot


def _allreduce_combine(o_part, stats):
    return pl.pallas_call(
        _allreduce_body,
        in_specs=[
            pl.BlockSpec(memory_space=pltpu.VMEM),
            pl.BlockSpec(memory_space=pltpu.VMEM),
        ],
        out_specs=pl.BlockSpec(memory_space=pltpu.VMEM),
        out_shape=jax.ShapeDtypeStruct((B, SQ, H, D), jnp.float32),
        scratch_shapes=[
            pltpu.VMEM((N_DEV, B, H, SQ, D), jnp.float32),
            pltpu.VMEM((N_DEV, B, H, SQ, 2), jnp.float32),
            pltpu.SemaphoreType.DMA((N_DEV - 1,)),
            pltpu.SemaphoreType.DMA((N_DEV - 1,)),
            pltpu.SemaphoreType.DMA((N_DEV - 1,)),
            pltpu.SemaphoreType.DMA((N_DEV - 1,)),
        ],
        compiler_params=pltpu.CompilerParams(collective_id=0),
    )(o_part, stats)


def kernel(Q, K, V):
    o_part, stats = _flash_partial(Q, K, V)
    return _allreduce_combine(o_part, stats)


# device time: 81304 ns/iter; 2.5837x vs baseline; 2.5837x over previous
import jax
import jax.numpy as jnp
from jax import lax
from jax.experimental import pallas as pl
from jax.experimental.pallas import tpu as pltpu

N_DEV = 4
B, SQ, H, D = 4, 32, 8, 128
BH = B * H
SCALE = D ** -0.5


def _flash_partial_body(q_ref, k_hbm, v_hbm, o_ref, st_ref,
                        kbuf, vbuf, ksems, vsems):
    def start_fetch(idx, slot):
        b, h = idx // H, idx % H
        pltpu.make_async_copy(
            k_hbm.at[b, :, h, :], kbuf.at[slot], ksems.at[slot]
        ).start()
        pltpu.make_async_copy(
            v_hbm.at[b, :, h, :], vbuf.at[slot], vsems.at[slot]
        ).start()

    def wait_fetch(idx, slot):
        b, h = idx // H, idx % H
        pltpu.make_async_copy(
            k_hbm.at[b, :, h, :], kbuf.at[slot], ksems.at[slot]
        ).wait()
        pltpu.make_async_copy(
            v_hbm.at[b, :, h, :], vbuf.at[slot], vsems.at[slot]
        ).wait()

    start_fetch(0, 0)
    for idx in range(BH):
        b, h = idx // H, idx % H
        slot = idx % 2
        if idx + 1 < BH:
            start_fetch(idx + 1, 1 - slot)
        wait_fetch(idx, slot)

        q = q_ref[b, :, h, :] * SCALE
        s = lax.dot_general(
            q, kbuf[slot], (((1,), (1,)), ((), ())),
            preferred_element_type=jnp.float32,
        )
        m = jnp.max(s, axis=1, keepdims=True)
        p = jnp.exp(s - m)
        l = jnp.sum(p, axis=1, keepdims=True)
        o_ref[b, h] = lax.dot_general(
            p, vbuf[slot], (((1,), (0,)), ((), ())),
            preferred_element_type=jnp.float32,
        )
        c = 2 * idx
        st_ref[:, c:c + 1] = m
        st_ref[:, c + 1:c + 2] = l


def _flash_partial(Q, K, V):
    skv = K.shape[1]
    return pl.pallas_call(
        _flash_partial_body,
        in_specs=[
            pl.BlockSpec(memory_space=pltpu.VMEM),
            pl.BlockSpec(memory_space=pltpu.MemorySpace.HBM),
            pl.BlockSpec(memory_space=pltpu.MemorySpace.HBM),
        ],
        out_specs=[
            pl.BlockSpec(memory_space=pltpu.VMEM),
            pl.BlockSpec(memory_space=pltpu.VMEM),
        ],
        out_shape=[
            jax.ShapeDtypeStruct((B, H, SQ, D), jnp.float32),
            jax.ShapeDtypeStruct((SQ, 2 * BH), jnp.float32),
        ],
        scratch_shapes=[
            pltpu.VMEM((2, skv, D), jnp.float32),
            pltpu.VMEM((2, skv, D), jnp.float32),
            pltpu.SemaphoreType.DMA((2,)),
            pltpu.SemaphoreType.DMA((2,)),
        ],
    )(Q, K, V)


def _allreduce_body(o_ref, st_ref, out_ref, o_comm, s_comm,
                    so_sems, ro_sems, ss_sems, rs_sems):
    my = lax.axis_index("i")
    left = lax.rem(my + N_DEV - 1, N_DEV)
    right = lax.rem(my + 1, N_DEV)

    o_comm[0] = o_ref[...]
    s_comm[0] = st_ref[...]

    barrier_sem = pltpu.get_barrier_semaphore()
    for nbr in (left, right):
        pl.semaphore_signal(
            barrier_sem, inc=1,
            device_id=(nbr,), device_id_type=pl.DeviceIdType.MESH,
        )
    pl.semaphore_wait(barrier_sem, 2)

    for h in range(N_DEV - 1):
        rdma_o = pltpu.make_async_remote_copy(
            src_ref=o_comm.at[h],
            dst_ref=o_comm.at[h + 1],
            send_sem=so_sems.at[h],
            recv_sem=ro_sems.at[h],
            device_id=(right,),
            device_id_type=pl.DeviceIdType.MESH,
        )
        rdma_s = pltpu.make_async_remote_copy(
            src_ref=s_comm.at[h],
            dst_ref=s_comm.at[h + 1],
            send_sem=ss_sems.at[h],
            recv_sem=rs_sems.at[h],
            device_id=(right,),
            device_id_type=pl.DeviceIdType.MESH,
        )
        rdma_o.start()
        rdma_s.start()
        rdma_o.wait()
        rdma_s.wait()

    for b in range(B):
        for hh in range(H):
            c = 2 * (b * H + hh)
            ms = [s_comm[j, :, c:c + 1] for j in range(N_DEV)]
            m_tot = ms[0]
            for j in range(1, N_DEV):
                m_tot = jnp.maximum(m_tot, ms[j])
            l_tot = jnp.zeros((SQ, 1), jnp.float32)
            o_tot = jnp.zeros((SQ, D), jnp.float32)
            for j in range(N_DEV):
                w = jnp.exp(ms[j] - m_tot)
                l_tot = l_tot + w * s_comm[j, :, c + 1:c + 2]
                o_tot = o_tot + w * o_comm[j, b, hh]
            out_ref[b, :, hh, :] = o_tot / l_tot


def _allreduce_combine(o_part, stats):
    return pl.pallas_call(
        _allreduce_body,
        in_specs=[
            pl.BlockSpec(memory_space=pltpu.VMEM),
            pl.BlockSpec(memory_space=pltpu.VMEM),
        ],
        out_specs=pl.BlockSpec(memory_space=pltpu.VMEM),
        out_shape=jax.ShapeDtypeStruct((B, SQ, H, D), jnp.float32),
        scratch_shapes=[
            pltpu.VMEM((N_DEV, B, H, SQ, D), jnp.float32),
            pltpu.VMEM((N_DEV, SQ, 2 * BH), jnp.float32),
            pltpu.SemaphoreType.DMA((N_DEV - 1,)),
            pltpu.SemaphoreType.DMA((N_DEV - 1,)),
            pltpu.SemaphoreType.DMA((N_DEV - 1,)),
            pltpu.SemaphoreType.DMA((N_DEV - 1,)),
        ],
        compiler_params=pltpu.CompilerParams(collective_id=0),
    )(o_part, stats)


def kernel(Q, K, V):
    o_part, stats = _flash_partial(Q, K, V)
    return _allreduce_combine(o_part, stats)


# device time: 62933 ns/iter; 3.3379x vs baseline; 1.2919x over previous
import jax
import jax.numpy as jnp
from jax import lax
from jax.experimental import pallas as pl
from jax.experimental.pallas import tpu as pltpu

N_DEV = 4
B, SQ, H, D = 4, 32, 8, 128
BH = B * H
H2 = H // 2
SCALE = D ** -0.5
NSLOTS = 4


def _stat_col(b, h):
    return 2 * (h * B + b)


def _flash_partial_body(q_ref, k_hbm, v_hbm, o_ref, st_ref,
                        kbuf, vbuf, ksems, vsems):
    def start_fetch(idx, slot):
        b, h = idx // H, idx % H
        pltpu.make_async_copy(
            k_hbm.at[b, :, h, :], kbuf.at[slot], ksems.at[slot]
        ).start()
        pltpu.make_async_copy(
            v_hbm.at[b, :, h, :], vbuf.at[slot], vsems.at[slot]
        ).start()

    def wait_fetch(idx, slot):
        b, h = idx // H, idx % H
        pltpu.make_async_copy(
            k_hbm.at[b, :, h, :], kbuf.at[slot], ksems.at[slot]
        ).wait()
        pltpu.make_async_copy(
            v_hbm.at[b, :, h, :], vbuf.at[slot], vsems.at[slot]
        ).wait()

    for j in range(NSLOTS - 1):
        start_fetch(j, j)
    for idx in range(BH):
        b, h = idx // H, idx % H
        slot = idx % NSLOTS
        nxt = idx + NSLOTS - 1
        if nxt < BH:
            start_fetch(nxt, nxt % NSLOTS)
        wait_fetch(idx, slot)

        q = q_ref[b, :, h, :] * SCALE
        s = lax.dot_general(
            q, kbuf[slot], (((1,), (1,)), ((), ())),
            preferred_element_type=jnp.float32,
        )
        m = jnp.max(s, axis=1, keepdims=True)
        p = jnp.exp(s - m)
        l = jnp.sum(p, axis=1, keepdims=True)
        o_ref[b, h] = lax.dot_general(
            p, vbuf[slot], (((1,), (0,)), ((), ())),
            preferred_element_type=jnp.float32,
        )
        c = _stat_col(b, h)
        st_ref[:, c:c + 1] = m
        st_ref[:, c + 1:c + 2] = l


def _flash_partial(Q, K, V):
    skv = K.shape[1]
    return pl.pallas_call(
        _flash_partial_body,
        in_specs=[
            pl.BlockSpec(memory_space=pltpu.MemorySpace.VMEM),
            pl.BlockSpec(memory_space=pltpu.MemorySpace.HBM),
            pl.BlockSpec(memory_space=pltpu.MemorySpace.HBM),
        ],
        out_specs=[
            pl.BlockSpec(memory_space=pltpu.MemorySpace.VMEM),
            pl.BlockSpec(memory_space=pltpu.MemorySpace.VMEM),
        ],
        out_shape=[
            jax.ShapeDtypeStruct((B, H, SQ, D), jnp.float32),
            jax.ShapeDtypeStruct((SQ, 2 * BH), jnp.float32),
        ],
        scratch_shapes=[
            pltpu.VMEM((NSLOTS, skv, D), jnp.float32),
            pltpu.VMEM((NSLOTS, skv, D), jnp.float32),
            pltpu.SemaphoreType.DMA((NSLOTS,)),
            pltpu.SemaphoreType.DMA((NSLOTS,)),
        ],
    )(Q, K, V)


def _allreduce_body(o_ref, st_ref, out_ref, o_comm, s_comm,
                    so1, ro1, ss1, rs1, so2, ro2, ss2, rs2):
    my = lax.axis_index("i")
    left = lax.rem(my + N_DEV - 1, N_DEV)
    right = lax.rem(my + 1, N_DEV)

    o_comm[0] = o_ref[...]
    s_comm[0] = st_ref[...]

    barrier_sem = pltpu.get_barrier_semaphore()
    for nbr in (left, right):
        pl.semaphore_signal(
            barrier_sem, inc=1,
            device_id=(nbr,), device_id_type=pl.DeviceIdType.MESH,
        )
    pl.semaphore_wait(barrier_sem, 2)

    def rdma(src, dst, send_sem, recv_sem, dev):
        return pltpu.make_async_remote_copy(
            src_ref=src, dst_ref=dst, send_sem=send_sem, recv_sem=recv_sem,
            device_id=(dev,), device_id_type=pl.DeviceIdType.MESH,
        )

    p1 = [
        rdma(o_comm.at[0], o_comm.at[1], so1.at[0], ro1.at[0], right),
        rdma(o_comm.at[0], o_comm.at[2], so1.at[1], ro1.at[1], left),
        rdma(s_comm.at[0], s_comm.at[1], ss1.at[0], rs1.at[0], right),
        rdma(s_comm.at[0], s_comm.at[2], ss1.at[1], rs1.at[1], left),
    ]
    for r in p1:
        r.start()
    for r in p1:
        r.wait()

    p2 = [
        rdma(o_comm.at[1, :, 0:H2], o_comm.at[3, :, 0:H2],
             so2.at[0], ro2.at[0], right),
        rdma(o_comm.at[2, :, H2:H], o_comm.at[3, :, H2:H],
             so2.at[1], ro2.at[1], left),
        rdma(s_comm.at[1], s_comm.at[3], ss2.at[0], rs2.at[0], right),
        rdma(s_comm.at[2], s_comm.at[3], ss2.at[1], rs2.at[1], left),
    ]
    for r in p2:
        r.start()
    for r in p2:
        r.wait()

    for b in range(B):
        for hh in range(H):
            c = _stat_col(b, hh)
            ms = [s_comm[j, :, c:c + 1] for j in range(N_DEV)]
            m_tot = ms[0]
            for j in range(1, N_DEV):
                m_tot = jnp.maximum(m_tot, ms[j])
            l_tot = jnp.zeros((SQ, 1), jnp.float32)
            o_tot = jnp.zeros((SQ, D), jnp.float32)
            for j in range(N_DEV):
                w = jnp.exp(ms[j] - m_tot)
                l_tot = l_tot + w * s_comm[j, :, c + 1:c + 2]
                o_tot = o_tot + w * o_comm[j, b, hh]
            out_ref[b, :, hh, :] = o_tot / l_tot


def _allreduce_combine(o_part, stats):
    return pl.pallas_call(
        _allreduce_body,
        in_specs=[
            pl.BlockSpec(memory_space=pltpu.MemorySpace.VMEM),
            pl.BlockSpec(memory_space=pltpu.MemorySpace.VMEM),
        ],
        out_specs=pl.BlockSpec(memory_space=pltpu.MemorySpace.VMEM),
        out_shape=jax.ShapeDtypeStruct((B, SQ, H, D), jnp.float32),
        scratch_shapes=[
            pltpu.VMEM((N_DEV, B, H, SQ, D), jnp.float32),
            pltpu.VMEM((N_DEV, SQ, 2 * BH), jnp.float32),
            pltpu.SemaphoreType.DMA((2,)),
            pltpu.SemaphoreType.DMA((2,)),
            pltpu.SemaphoreType.DMA((2,)),
            pltpu.SemaphoreType.DMA((2,)),
            pltpu.SemaphoreType.DMA((2,)),
            pltpu.SemaphoreType.DMA((2,)),
            pltpu.SemaphoreType.DMA((2,)),
            pltpu.SemaphoreType.DMA((2,)),
        ],
        compiler_params=pltpu.CompilerParams(collective_id=0),
    )(o_part, stats)


def kernel(Q, K, V):
    o_part, stats = _flash_partial(Q, K, V)
    return _allreduce_combine(o_part, stats)


# device time: 58017 ns/iter; 3.6208x vs baseline; 1.0847x over previous
import jax
import jax.numpy as jnp
from jax import lax
from jax.experimental import pallas as pl
from jax.experimental.pallas import tpu as pltpu

N_DEV = 4
B, SQ, H, D = 4, 32, 8, 128
BH = B * H
H2 = H // 2
SCALE = D ** -0.5
NSLOTS = 4


def _stat_col(b, h):
    return 2 * (h * B + b)


def _flash_partial_body(q_ref, k_hbm, v_hbm, o_ref, st_ref,
                        kbuf, vbuf, ksems, vsems):
    def start_fetch(idx, slot):
        b, h = idx // H, idx % H
        pltpu.make_async_copy(
            k_hbm.at[b, :, h, :], kbuf.at[slot], ksems.at[slot]
        ).start()
        pltpu.make_async_copy(
            v_hbm.at[b, :, h, :], vbuf.at[slot], vsems.at[slot]
        ).start()

    def wait_fetch(idx, slot):
        b, h = idx // H, idx % H
        pltpu.make_async_copy(
            k_hbm.at[b, :, h, :], kbuf.at[slot], ksems.at[slot]
        ).wait()
        pltpu.make_async_copy(
            v_hbm.at[b, :, h, :], vbuf.at[slot], vsems.at[slot]
        ).wait()

    for j in range(NSLOTS - 1):
        start_fetch(j, j)
    for idx in range(BH):
        b, h = idx // H, idx % H
        slot = idx % NSLOTS
        nxt = idx + NSLOTS - 1
        if nxt < BH:
            start_fetch(nxt, nxt % NSLOTS)
        wait_fetch(idx, slot)

        q = q_ref[b, :, h, :] * SCALE
        s = lax.dot_general(
            q, kbuf[slot], (((1,), (1,)), ((), ())),
            preferred_element_type=jnp.float32,
        )
        m = jnp.max(s, axis=1, keepdims=True)
        p = jnp.exp(s - m)
        l = jnp.sum(p, axis=1, keepdims=True)
        o_ref[b, h] = lax.dot_general(
            p, vbuf[slot], (((1,), (0,)), ((), ())),
            preferred_element_type=jnp.float32,
        )
        c = _stat_col(b, h)
        st_ref[:, c:c + 1] = m
        st_ref[:, c + 1:c + 2] = l


def _flash_partial(Q, K, V):
    skv = K.shape[1]
    return pl.pallas_call(
        _flash_partial_body,
        in_specs=[
            pl.BlockSpec(memory_space=pltpu.MemorySpace.VMEM),
            pl.BlockSpec(memory_space=pltpu.MemorySpace.HBM),
            pl.BlockSpec(memory_space=pltpu.MemorySpace.HBM),
        ],
        out_specs=[
            pl.BlockSpec(memory_space=pltpu.MemorySpace.VMEM),
            pl.BlockSpec(memory_space=pltpu.MemorySpace.VMEM),
        ],
        out_shape=[
            jax.ShapeDtypeStruct((B, H, SQ, D), jnp.float32),
            jax.ShapeDtypeStruct((SQ, 2 * BH), jnp.float32),
        ],
        scratch_shapes=[
            pltpu.VMEM((NSLOTS, skv, D), jnp.float32),
            pltpu.VMEM((NSLOTS, skv, D), jnp.float32),
            pltpu.SemaphoreType.DMA((NSLOTS,)),
            pltpu.SemaphoreType.DMA((NSLOTS,)),
        ],
    )(Q, K, V)


def _allreduce_body(o_ref, st_ref, out_ref, o_comm, s_comm,
                    so1, ro1, ss1, rs1, so2, ro2, ss2, rs2):
    my = lax.axis_index("i")
    left = lax.rem(my + N_DEV - 1, N_DEV)
    right = lax.rem(my + 1, N_DEV)

    o_comm[0] = o_ref[...]
    s_comm[0] = st_ref[...]

    barrier_sem = pltpu.get_barrier_semaphore()
    for nbr in (left, right):
        pl.semaphore_signal(
            barrier_sem, inc=1,
            device_id=(nbr,), device_id_type=pl.DeviceIdType.MESH,
        )
    pl.semaphore_wait(barrier_sem, 2)

    def rdma(src, dst, send_sem, recv_sem, dev):
        return pltpu.make_async_remote_copy(
            src_ref=src, dst_ref=dst, send_sem=send_sem, recv_sem=recv_sem,
            device_id=(dev,), device_id_type=pl.DeviceIdType.MESH,
        )

    p1 = [
        rdma(o_comm.at[0], o_comm.at[1], so1.at[0], ro1.at[0], right),
        rdma(o_comm.at[0], o_comm.at[2], so1.at[1], ro1.at[1], left),
        rdma(s_comm.at[0], s_comm.at[1], ss1.at[0], rs1.at[0], right),
        rdma(s_comm.at[0], s_comm.at[2], ss1.at[1], rs1.at[1], left),
    ]
    for r in p1:
        r.start()
    for r in p1:
        r.wait()

    p2 = [
        rdma(o_comm.at[1, :, 0:H2], o_comm.at[3, :, 0:H2],
             so2.at[0], ro2.at[0], right),
        rdma(o_comm.at[2, :, H2:H], o_comm.at[3, :, H2:H],
             so2.at[1], ro2.at[1], left),
        rdma(s_comm.at[1], s_comm.at[3], ss2.at[0], rs2.at[0], right),
        rdma(s_comm.at[2], s_comm.at[3], ss2.at[1], rs2.at[1], left),
    ]
    for r in p2:
        r.start()
    for r in p2:
        r.wait()

    for b in range(B):
        for hh in range(H):
            c = _stat_col(b, hh)
            ms = [s_comm[j, :, c:c + 1] for j in range(N_DEV)]
            m_tot = ms[0]
            for j in range(1, N_DEV):
                m_tot = jnp.maximum(m_tot, ms[j])
            l_tot = jnp.zeros((SQ, 1), jnp.float32)
            o_tot = jnp.zeros((SQ, D), jnp.float32)
            for j in range(N_DEV):
                w = jnp.exp(ms[j] - m_tot)
                l_tot = l_tot + w * s_comm[j, :, c + 1:c + 2]
                o_tot = o_tot + w * o_comm[j, b, hh]
            out_ref[b, :, hh, :] = o_tot / l_tot


def _allreduce_combine(o_part, stats):
    return pl.pallas_call(
        _allreduce_body,
        in_specs=[
            pl.BlockSpec(memory_space=pltpu.MemorySpace.VMEM),
            pl.BlockSpec(memory_space=pltpu.MemorySpace.VMEM),
        ],
        out_specs=pl.BlockSpec(memory_space=pltpu.MemorySpace.VMEM),
        out_shape=jax.ShapeDtypeStruct((B, SQ, H, D), jnp.float32),
        scratch_shapes=[
            pltpu.VMEM((N_DEV, B, H, SQ, D), jnp.float32),
            pltpu.VMEM((N_DEV, SQ, 2 * BH), jnp.float32),
            pltpu.SemaphoreType.DMA((2,)),
            pltpu.SemaphoreType.DMA((2,)),
            pltpu.SemaphoreType.DMA((2,)),
            pltpu.SemaphoreType.DMA((2,)),
            pltpu.SemaphoreType.DMA((2,)),
            pltpu.SemaphoreType.DMA((2,)),
            pltpu.SemaphoreType.DMA((2,)),
            pltpu.SemaphoreType.DMA((2,)),
        ],
        compiler_params=pltpu.CompilerParams(collective_id=0),
    )(o_part, stats)


def _fused_body(q_ref, k_hbm, v_hbm, out_ref,
                kbuf, vbuf, ksems, vsems, o_comm, s_comm,
                so1, ro1, ss1, rs1, so2, ro2, ss2, rs2):
    my = lax.axis_index("i")
    left = lax.rem(my + N_DEV - 1, N_DEV)
    right = lax.rem(my + 1, N_DEV)

    barrier_sem = pltpu.get_barrier_semaphore()
    for nbr in (left, right):
        pl.semaphore_signal(
            barrier_sem, inc=1,
            device_id=(nbr,), device_id_type=pl.DeviceIdType.MESH,
        )
    pl.semaphore_wait(barrier_sem, 2)

    def rdma(src, dst, send_sem, recv_sem, dev):
        return pltpu.make_async_remote_copy(
            src_ref=src, dst_ref=dst, send_sem=send_sem, recv_sem=recv_sem,
            device_id=(dev,), device_id_type=pl.DeviceIdType.MESH,
        )

    p1r = [rdma(o_comm.at[0, b], o_comm.at[1, b], so1.at[0, b], ro1.at[0, b],
                right) for b in range(B)]
    p1l = [rdma(o_comm.at[0, b], o_comm.at[2, b], so1.at[1, b], ro1.at[1, b],
                left) for b in range(B)]
    p2r = [rdma(o_comm.at[1, b, 0:H2], o_comm.at[3, b, 0:H2],
                so2.at[0, b], ro2.at[0, b], right) for b in range(B)]
    p2l = [rdma(o_comm.at[2, b, H2:H], o_comm.at[3, b, H2:H],
                so2.at[1, b], ro2.at[1, b], left) for b in range(B)]
    s1r = rdma(s_comm.at[0], s_comm.at[1], ss1.at[0], rs1.at[0], right)
    s1l = rdma(s_comm.at[0], s_comm.at[2], ss1.at[1], rs1.at[1], left)
    s2r = rdma(s_comm.at[1], s_comm.at[3], ss2.at[0], rs2.at[0], right)
    s2l = rdma(s_comm.at[2], s_comm.at[3], ss2.at[1], rs2.at[1], left)

    def start_fetch(idx, slot):
        b, h = idx // H, idx % H
        pltpu.make_async_copy(
            k_hbm.at[b, :, h, :], kbuf.at[slot], ksems.at[slot]
        ).start()
        pltpu.make_async_copy(
            v_hbm.at[b, :, h, :], vbuf.at[slot], vsems.at[slot]
        ).start()

    def wait_fetch(idx, slot):
        b, h = idx // H, idx % H
        pltpu.make_async_copy(
            k_hbm.at[b, :, h, :], kbuf.at[slot], ksems.at[slot]
        ).wait()
        pltpu.make_async_copy(
            v_hbm.at[b, :, h, :], vbuf.at[slot], vsems.at[slot]
        ).wait()

    for j in range(NSLOTS - 1):
        start_fetch(j, j)
    for idx in range(BH):
        b, h = idx // H, idx % H
        slot = idx % NSLOTS
        nxt = idx + NSLOTS - 1
        if nxt < BH:
            start_fetch(nxt, nxt % NSLOTS)
        wait_fetch(idx, slot)

        q = q_ref[b, :, h, :] * SCALE
        s = lax.dot_general(
            q, kbuf[slot], (((1,), (1,)), ((), ())),
            preferred_element_type=jnp.float32,
        )
        m = jnp.max(s, axis=1, keepdims=True)
        p = jnp.exp(s - m)
        l = jnp.sum(p, axis=1, keepdims=True)
        o_comm[0, b, h] = lax.dot_general(
            p, vbuf[slot], (((1,), (0,)), ((), ())),
            preferred_element_type=jnp.float32,
        )
        c = _stat_col(b, h)
        s_comm[0, :, c:c + 1] = m
        s_comm[0, :, c + 1:c + 2] = l

        if h == H - 1:
            p1r[b].start()
            p1l[b].start()

    s1r.start()
    s1l.start()

    for b in range(B):
        p1r[b].wait_recv()
        p1l[b].wait_recv()
        p2r[b].start()
        p2l[b].start()
    s1r.wait_recv()
    s1l.wait_recv()
    s2r.start()
    s2l.start()

    for b in range(B):
        p2r[b].wait_recv()
        p2l[b].wait_recv()
    s2r.wait_recv()
    s2l.wait_recv()

    for b in range(B):
        for hh in range(H):
            c = _stat_col(b, hh)
            ms = [s_comm[j, :, c:c + 1] for j in range(N_DEV)]
            m_tot = ms[0]
            for j in range(1, N_DEV):
                m_tot = jnp.maximum(m_tot, ms[j])
            l_tot = jnp.zeros((SQ, 1), jnp.float32)
            o_tot = jnp.zeros((SQ, D), jnp.float32)
            for j in range(N_DEV):
                w = jnp.exp(ms[j] - m_tot)
                l_tot = l_tot + w * s_comm[j, :, c + 1:c + 2]
                o_tot = o_tot + w * o_comm[j, b, hh]
            out_ref[b, :, hh, :] = o_tot / l_tot

    for b in range(B):
        p1r[b].wait_send()
        p1l[b].wait_send()
        p2r[b].wait_send()
        p2l[b].wait_send()
    for r in (s1r, s1l, s2r, s2l):
        r.wait_send()


def _fused(Q, K, V):
    skv = K.shape[1]
    return pl.pallas_call(
        _fused_body,
        in_specs=[
            pl.BlockSpec(memory_space=pltpu.MemorySpace.VMEM),
            pl.BlockSpec(memory_space=pltpu.MemorySpace.HBM),
            pl.BlockSpec(memory_space=pltpu.MemorySpace.HBM),
        ],
        out_specs=pl.BlockSpec(memory_space=pltpu.MemorySpace.VMEM),
        out_shape=jax.ShapeDtypeStruct((B, SQ, H, D), jnp.float32),
        scratch_shapes=[
            pltpu.VMEM((NSLOTS, skv, D), jnp.float32),
            pltpu.VMEM((NSLOTS, skv, D), jnp.float32),
            pltpu.SemaphoreType.DMA((NSLOTS,)),
            pltpu.SemaphoreType.DMA((NSLOTS,)),
            pltpu.VMEM((N_DEV, B, H, SQ, D), jnp.float32),
            pltpu.VMEM((N_DEV, SQ, 2 * BH), jnp.float32),
            pltpu.SemaphoreType.DMA((2, B)),
            pltpu.SemaphoreType.DMA((2, B)),
            pltpu.SemaphoreType.DMA((2,)),
            pltpu.SemaphoreType.DMA((2,)),
            pltpu.SemaphoreType.DMA((2, B)),
            pltpu.SemaphoreType.DMA((2, B)),
            pltpu.SemaphoreType.DMA((2,)),
            pltpu.SemaphoreType.DMA((2,)),
        ],
        compiler_params=pltpu.CompilerParams(collective_id=0),
    )(Q, K, V)


def kernel(Q, K, V):
    return _fused(Q, K, V)


# device time: 54358 ns/iter; 3.8645x vs baseline; 1.0673x over previous
import jax
import jax.numpy as jnp
from jax import lax
from jax.experimental import pallas as pl
from jax.experimental.pallas import tpu as pltpu

N_DEV = 4
B, SQ, H, D = 4, 32, 8, 128
BH = B * H
H2 = H // 2
SCALE = D ** -0.5
NSLOTS = 4


def _stat_col(b, h):
    return 2 * (h * B + b)


def _flash_partial_body(q_ref, k_hbm, v_hbm, o_ref, st_ref,
                        kbuf, vbuf, ksems, vsems):
    def start_fetch(idx, slot):
        b, h = idx // H, idx % H
        pltpu.make_async_copy(
            k_hbm.at[b, :, h, :], kbuf.at[slot], ksems.at[slot]
        ).start()
        pltpu.make_async_copy(
            v_hbm.at[b, :, h, :], vbuf.at[slot], vsems.at[slot]
        ).start()

    def wait_fetch(idx, slot):
        b, h = idx // H, idx % H
        pltpu.make_async_copy(
            k_hbm.at[b, :, h, :], kbuf.at[slot], ksems.at[slot]
        ).wait()
        pltpu.make_async_copy(
            v_hbm.at[b, :, h, :], vbuf.at[slot], vsems.at[slot]
        ).wait()

    for j in range(NSLOTS - 1):
        start_fetch(j, j)
    for idx in range(BH):
        b, h = idx // H, idx % H
        slot = idx % NSLOTS
        nxt = idx + NSLOTS - 1
        if nxt < BH:
            start_fetch(nxt, nxt % NSLOTS)
        wait_fetch(idx, slot)

        q = q_ref[b, :, h, :] * SCALE
        s = lax.dot_general(
            q, kbuf[slot], (((1,), (1,)), ((), ())),
            preferred_element_type=jnp.float32,
        )
        m = jnp.max(s, axis=1, keepdims=True)
        p = jnp.exp(s - m)
        l = jnp.sum(p, axis=1, keepdims=True)
        o_ref[b, h] = lax.dot_general(
            p, vbuf[slot], (((1,), (0,)), ((), ())),
            preferred_element_type=jnp.float32,
        )
        c = _stat_col(b, h)
        st_ref[:, c:c + 1] = m
        st_ref[:, c + 1:c + 2] = l


def _flash_partial(Q, K, V):
    skv = K.shape[1]
    return pl.pallas_call(
        _flash_partial_body,
        in_specs=[
            pl.BlockSpec(memory_space=pltpu.MemorySpace.VMEM),
            pl.BlockSpec(memory_space=pltpu.MemorySpace.HBM),
            pl.BlockSpec(memory_space=pltpu.MemorySpace.HBM),
        ],
        out_specs=[
            pl.BlockSpec(memory_space=pltpu.MemorySpace.VMEM),
            pl.BlockSpec(memory_space=pltpu.MemorySpace.VMEM),
        ],
        out_shape=[
            jax.ShapeDtypeStruct((B, H, SQ, D), jnp.float32),
            jax.ShapeDtypeStruct((SQ, 2 * BH), jnp.float32),
        ],
        scratch_shapes=[
            pltpu.VMEM((NSLOTS, skv, D), jnp.float32),
            pltpu.VMEM((NSLOTS, skv, D), jnp.float32),
            pltpu.SemaphoreType.DMA((NSLOTS,)),
            pltpu.SemaphoreType.DMA((NSLOTS,)),
        ],
    )(Q, K, V)


def _allreduce_body(o_ref, st_ref, out_ref, o_comm, s_comm,
                    so1, ro1, ss1, rs1, so2, ro2, ss2, rs2):
    my = lax.axis_index("i")
    left = lax.rem(my + N_DEV - 1, N_DEV)
    right = lax.rem(my + 1, N_DEV)

    o_comm[0] = o_ref[...]
    s_comm[0] = st_ref[...]

    barrier_sem = pltpu.get_barrier_semaphore()
    for nbr in (left, right):
        pl.semaphore_signal(
            barrier_sem, inc=1,
            device_id=(nbr,), device_id_type=pl.DeviceIdType.MESH,
        )
    pl.semaphore_wait(barrier_sem, 2)

    def rdma(src, dst, send_sem, recv_sem, dev):
        return pltpu.make_async_remote_copy(
            src_ref=src, dst_ref=dst, send_sem=send_sem, recv_sem=recv_sem,
            device_id=(dev,), device_id_type=pl.DeviceIdType.MESH,
        )

    p1 = [
        rdma(o_comm.at[0], o_comm.at[1], so1.at[0], ro1.at[0], right),
        rdma(o_comm.at[0], o_comm.at[2], so1.at[1], ro1.at[1], left),
        rdma(s_comm.at[0], s_comm.at[1], ss1.at[0], rs1.at[0], right),
        rdma(s_comm.at[0], s_comm.at[2], ss1.at[1], rs1.at[1], left),
    ]
    for r in p1:
        r.start()
    for r in p1:
        r.wait()

    p2 = [
        rdma(o_comm.at[1, :, 0:H2], o_comm.at[3, :, 0:H2],
             so2.at[0], ro2.at[0], right),
        rdma(o_comm.at[2, :, H2:H], o_comm.at[3, :, H2:H],
             so2.at[1], ro2.at[1], left),
        rdma(s_comm.at[1], s_comm.at[3], ss2.at[0], rs2.at[0], right),
        rdma(s_comm.at[2], s_comm.at[3], ss2.at[1], rs2.at[1], left),
    ]
    for r in p2:
        r.start()
    for r in p2:
        r.wait()

    for b in range(B):
        for hh in range(H):
            c = _stat_col(b, hh)
            ms = [s_comm[j, :, c:c + 1] for j in range(N_DEV)]
            m_tot = ms[0]
            for j in range(1, N_DEV):
                m_tot = jnp.maximum(m_tot, ms[j])
            l_tot = jnp.zeros((SQ, 1), jnp.float32)
            o_tot = jnp.zeros((SQ, D), jnp.float32)
            for j in range(N_DEV):
                w = jnp.exp(ms[j] - m_tot)
                l_tot = l_tot + w * s_comm[j, :, c + 1:c + 2]
                o_tot = o_tot + w * o_comm[j, b, hh]
            out_ref[b, :, hh, :] = o_tot / l_tot


def _allreduce_combine(o_part, stats):
    return pl.pallas_call(
        _allreduce_body,
        in_specs=[
            pl.BlockSpec(memory_space=pltpu.MemorySpace.VMEM),
            pl.BlockSpec(memory_space=pltpu.MemorySpace.VMEM),
        ],
        out_specs=pl.BlockSpec(memory_space=pltpu.MemorySpace.VMEM),
        out_shape=jax.ShapeDtypeStruct((B, SQ, H, D), jnp.float32),
        scratch_shapes=[
            pltpu.VMEM((N_DEV, B, H, SQ, D), jnp.float32),
            pltpu.VMEM((N_DEV, SQ, 2 * BH), jnp.float32),
            pltpu.SemaphoreType.DMA((2,)),
            pltpu.SemaphoreType.DMA((2,)),
            pltpu.SemaphoreType.DMA((2,)),
            pltpu.SemaphoreType.DMA((2,)),
            pltpu.SemaphoreType.DMA((2,)),
            pltpu.SemaphoreType.DMA((2,)),
            pltpu.SemaphoreType.DMA((2,)),
            pltpu.SemaphoreType.DMA((2,)),
        ],
        compiler_params=pltpu.CompilerParams(collective_id=0),
    )(o_part, stats)


def _fused_body(q_ref, k_hbm, v_hbm, out_ref,
                kbuf, vbuf, ksems, vsems, o_comm,
                so1, ro1, so2, ro2):
    my = lax.axis_index("i")
    left = lax.rem(my + N_DEV - 1, N_DEV)
    right = lax.rem(my + 1, N_DEV)

    barrier_sem = pltpu.get_barrier_semaphore()
    for nbr in (left, right):
        pl.semaphore_signal(
            barrier_sem, inc=1,
            device_id=(nbr,), device_id_type=pl.DeviceIdType.MESH,
        )
    pl.semaphore_wait(barrier_sem, 2)

    def rdma(src, dst, send_sem, recv_sem, dev):
        return pltpu.make_async_remote_copy(
            src_ref=src, dst_ref=dst, send_sem=send_sem, recv_sem=recv_sem,
            device_id=(dev,), device_id_type=pl.DeviceIdType.MESH,
        )

    p1r = [rdma(o_comm.at[0, b], o_comm.at[1, b], so1.at[0, b], ro1.at[0, b],
                right) for b in range(B)]
    p1l = [rdma(o_comm.at[0, b], o_comm.at[2, b], so1.at[1, b], ro1.at[1, b],
                left) for b in range(B)]
    p2r = [rdma(o_comm.at[1, b, 0:H2], o_comm.at[3, b, 0:H2],
                so2.at[0, b], ro2.at[0, b], right) for b in range(B)]
    p2l = [rdma(o_comm.at[2, b, H2:H + 1], o_comm.at[3, b, H2:H + 1],
                so2.at[1, b], ro2.at[1, b], left) for b in range(B)]

    def start_fetch(idx, slot):
        b, h = idx // H, idx % H
        pltpu.make_async_copy(
            k_hbm.at[b, :, h, :], kbuf.at[slot], ksems.at[slot]
        ).start()
        pltpu.make_async_copy(
            v_hbm.at[b, :, h, :], vbuf.at[slot], vsems.at[slot]
        ).start()

    def wait_fetch(idx, slot):
        b, h = idx // H, idx % H
        pltpu.make_async_copy(
            k_hbm.at[b, :, h, :], kbuf.at[slot], ksems.at[slot]
        ).wait()
        pltpu.make_async_copy(
            v_hbm.at[b, :, h, :], vbuf.at[slot], vsems.at[slot]
        ).wait()

    def combine_chunk(b):
        for hh in range(H):
            ms = [o_comm[j, b, H, :, 2 * hh:2 * hh + 1] for j in range(N_DEV)]
            m_tot = ms[0]
            for j in range(1, N_DEV):
                m_tot = jnp.maximum(m_tot, ms[j])
            l_tot = jnp.zeros((SQ, 1), jnp.float32)
            o_tot = jnp.zeros((SQ, D), jnp.float32)
            for j in range(N_DEV):
                w = jnp.exp(ms[j] - m_tot)
                l_tot = l_tot + w * o_comm[j, b, H, :, 2 * hh + 1:2 * hh + 2]
                o_tot = o_tot + w * o_comm[j, b, hh]
            out_ref[b, :, hh, :] = o_tot / l_tot

    for j in range(NSLOTS - 1):
        start_fetch(j, j)
    for idx in range(BH):
        b, h = idx // H, idx % H
        slot = idx % NSLOTS
        nxt = idx + NSLOTS - 1
        if nxt < BH:
            start_fetch(nxt, nxt % NSLOTS)
        wait_fetch(idx, slot)

        q = q_ref[b, :, h, :] * SCALE
        s = lax.dot_general(
            q, kbuf[slot], (((1,), (1,)), ((), ())),
            preferred_element_type=jnp.float32,
        )
        m = jnp.max(s, axis=1, keepdims=True)
        p = jnp.exp(s - m)
        l = jnp.sum(p, axis=1, keepdims=True)
        o_comm[0, b, h] = lax.dot_general(
            p, vbuf[slot], (((1,), (0,)), ((), ())),
            preferred_element_type=jnp.float32,
        )
        o_comm[0, b, H, :, 2 * h:2 * h + 1] = m
        o_comm[0, b, H, :, 2 * h + 1:2 * h + 2] = l

        if h == H - 1:
            p1r[b].start()
            p1l[b].start()
            if b >= 1:
                p1r[b - 1].wait_recv()
                p1l[b - 1].wait_recv()
                p2r[b - 1].start()
                p2l[b - 1].start()
            if b >= 2:
                p2r[b - 2].wait_recv()
                p2l[b - 2].wait_recv()
                combine_chunk(b - 2)

    p1r[B - 1].wait_recv()
    p1l[B - 1].wait_recv()
    p2r[B - 1].start()
    p2l[B - 1].start()
    p2r[B - 2].wait_recv()
    p2l[B - 2].wait_recv()
    combine_chunk(B - 2)
    p2r[B - 1].wait_recv()
    p2l[B - 1].wait_recv()
    combine_chunk(B - 1)

    for b in range(B):
        p1r[b].wait_send()
        p1l[b].wait_send()
        p2r[b].wait_send()
        p2l[b].wait_send()


def _fused(Q, K, V):
    skv = K.shape[1]
    return pl.pallas_call(
        _fused_body,
        in_specs=[
            pl.BlockSpec(memory_space=pltpu.MemorySpace.VMEM),
            pl.BlockSpec(memory_space=pltpu.MemorySpace.HBM),
            pl.BlockSpec(memory_space=pltpu.MemorySpace.HBM),
        ],
        out_specs=pl.BlockSpec(memory_space=pltpu.MemorySpace.VMEM),
        out_shape=jax.ShapeDtypeStruct((B, SQ, H, D), jnp.float32),
        scratch_shapes=[
            pltpu.VMEM((NSLOTS, skv, D), jnp.float32),
            pltpu.VMEM((NSLOTS, skv, D), jnp.float32),
            pltpu.SemaphoreType.DMA((NSLOTS,)),
            pltpu.SemaphoreType.DMA((NSLOTS,)),
            pltpu.VMEM((N_DEV, B, H + 1, SQ, D), jnp.float32),
            pltpu.SemaphoreType.DMA((2, B)),
            pltpu.SemaphoreType.DMA((2, B)),
            pltpu.SemaphoreType.DMA((2, B)),
            pltpu.SemaphoreType.DMA((2, B)),
        ],
        compiler_params=pltpu.CompilerParams(collective_id=0),
    )(Q, K, V)


def kernel(Q, K, V):
    return _fused(Q, K, V)
